# Initial kernel scaffold; baseline (speedup 1.0000x reference)
import jax
import jax.numpy as jnp
from jax.experimental import pallas as pl


def kernel(query_encoding, entity_embedding):
    raise NotImplementedError("write your pallas kernel here")


# speedup vs baseline: 1.9382x; 1.9382x over previous
import jax
import jax.numpy as jnp
from jax.experimental import pallas as pl
from jax.experimental.pallas import tpu as pltpu

_B = 1024
_D = 128
_NE = 100000
_W = 2048
_NB = 49
_RUN = 128
_K = 10

_NEG = float("-inf")
_IMAX = 2**31 - 1


def _topk_kernel(q_ref, e_ref, outv_ref, outi_ref, c_ref, ci_ref):
    j = pl.program_id(0)

    @pl.when(j == 0)
    def _init():
        c_ref[:, :_RUN] = jnp.full((_B, _RUN), _NEG, jnp.float32)
        ci_ref[:, :_RUN] = jnp.full((_B, _RUN), _IMAX, jnp.int32)

    qb = q_ref[:, :].astype(jnp.bfloat16)
    eb = e_ref[:, :].astype(jnp.bfloat16)
    s = jax.lax.dot_general(qb, eb, (((1,), (1,)), ((), ())),
                            preferred_element_type=jnp.float32)
    gcol = j * _W + jax.lax.broadcasted_iota(jnp.int32, (_B, _W), 1)
    s = jnp.where(gcol < _NE, s, _NEG)
    c_ref[:, _RUN:] = s
    ci_ref[:, _RUN:] = gcol

    c = c_ref[:, :]
    ci = ci_ref[:, :]
    vals, idxs = [], []
    for _ in range(_K):
        m = jnp.max(c, axis=1, keepdims=True)
        eq = c == m
        gi = jnp.min(jnp.where(eq, ci, _IMAX), axis=1, keepdims=True)
        vals.append(m)
        idxs.append(gi)
        c = jnp.where(eq & (ci == gi), _NEG, c)
    v10 = jnp.concatenate(vals, axis=1)
    i10 = jnp.concatenate(idxs, axis=1)
    c_ref[:, :_K] = v10
    ci_ref[:, :_K] = i10
    outv_ref[:, :] = v10
    outi_ref[:, :] = i10


def kernel(query_encoding, entity_embedding):
    e_pad = jnp.pad(entity_embedding, ((0, _NB * _W - _NE), (0, 0)))
    vals, idx = pl.pallas_call(
        _topk_kernel,
        grid=(_NB,),
        in_specs=[
            pl.BlockSpec((_B, _D), lambda j: (0, 0)),
            pl.BlockSpec((_W, _D), lambda j: (j, 0)),
        ],
        out_specs=(
            pl.BlockSpec((_B, _K), lambda j: (0, 0)),
            pl.BlockSpec((_B, _K), lambda j: (0, 0)),
        ),
        out_shape=(
            jax.ShapeDtypeStruct((_B, _K), jnp.float32),
            jax.ShapeDtypeStruct((_B, _K), jnp.int32),
        ),
        scratch_shapes=[
            pltpu.VMEM((_B, _RUN + _W), jnp.float32),
            pltpu.VMEM((_B, _RUN + _W), jnp.int32),
        ],
    )(query_encoding, e_pad)
    return (vals, idx)


# speedup vs baseline: 4.1768x; 2.1550x over previous
import functools

import jax
import jax.numpy as jnp
from jax.experimental import pallas as pl
from jax.experimental.pallas import tpu as pltpu

_B = 1024
_RB = 128
_D = 128
_NE = 100000
_W = 2048
_NB = 49
_NC = _W // 128
_K = 10

_NEG = float("-inf")
_IMAX = 2**31 - 1


def _oddeven_sort_pairs(n):
    pairs = []

    def merge(lo, n_, r):
        step = r * 2
        if step < n_:
            merge(lo, n_, step)
            merge(lo + r, n_, step)
            for i in range(lo + r, lo + n_ - r, step):
                pairs.append((i, i + r))
        else:
            pairs.append((lo, lo + r))

    def sort(lo, n_):
        if n_ > 1:
            m = n_ // 2
            sort(lo, m)
            sort(lo + m, m)
            merge(lo, n_, 1)

    sort(0, n)
    return pairs


def _bitonic_merge16_top10_pairs():
    pairs = []
    for d in (8, 4, 2, 1):
        for i in range(16):
            if (i // d) % 2 == 0 and i + d < 16:
                if d == 2 and i >= 12:
                    continue
                if d == 1 and i >= 10:
                    continue
                pairs.append((i, i + d))
    return pairs


_SORT16 = _oddeven_sort_pairs(_NC)
_MERGEK = _bitonic_merge16_top10_pairs()


def _ce(v, ids, a, b):
    c = v[b] > v[a]
    hi = jnp.maximum(v[a], v[b])
    lo = jnp.minimum(v[a], v[b])
    ihi = jnp.where(c, ids[b], ids[a])
    ilo = jnp.where(c, ids[a], ids[b])
    v[a], v[b] = hi, lo
    ids[a], ids[b] = ihi, ilo


def _topk_kernel(q_ref, e_ref, outv_ref, outi_ref, rv_ref, rn_ref):
    j = pl.program_id(1)

    @pl.when(j == 0)
    def _init():
        rv_ref[:, :] = jnp.full((_RB, _K * 128), _NEG, jnp.float32)
        rn_ref[:, :] = jnp.zeros((_RB, _K * 128), jnp.int32)

    qb = q_ref[:, :].astype(jnp.bfloat16)
    eb = e_ref[:, :].astype(jnp.bfloat16)
    s = jax.lax.dot_general(qb, eb, (((1,), (1,)), ((), ())),
                            preferred_element_type=jnp.float32)

    lane = jax.lax.broadcasted_iota(jnp.int32, (_RB, 128), 1)
    v = []
    ids = []
    for c in range(_NC):
        vc = s[:, c * 128:(c + 1) * 128]
        nc = j * _NC + c
        if (c + 1) * 128 > _NE - (_NB - 1) * _W:
            gcol = nc * 128 + lane
            vc = jnp.where(gcol < _NE, vc, _NEG)
        v.append(vc)
        ids.append(jnp.full((_RB, 128), nc, jnp.int32))

    for (a, b) in _SORT16:
        _ce(v, ids, a, b)

    w = []
    wi = []
    for i in range(_NC):
        if _NC - 1 - i >= _K:
            w.append(v[i])
            wi.append(ids[i])
        else:
            bv = rv_ref[:, (_NC - 1 - i) * 128:(_NC - i) * 128]
            bn = rn_ref[:, (_NC - 1 - i) * 128:(_NC - i) * 128]
            c = bv > v[i]
            w.append(jnp.maximum(v[i], bv))
            wi.append(jnp.where(c, bn, ids[i]))
    for (a, b) in _MERGEK:
        _ce(w, wi, a, b)
    for i in range(_K):
        rv_ref[:, i * 128:(i + 1) * 128] = w[i]
        rn_ref[:, i * 128:(i + 1) * 128] = wi[i]

    @pl.when(j == _NB - 1)
    def _finalize():
        levels = [w[i] for i in range(_K)]
        gidx = [wi[i] * 128 + lane for i in range(_K)]
        out_v = []
        out_i = []
        for k in range(_K):
            nl = k + 1
            m = levels[0].max(axis=1, keepdims=True)
            for l in range(1, nl):
                m = jnp.maximum(m, levels[l].max(axis=1, keepdims=True))
            eqs = [levels[l] == m for l in range(nl)]
            g = jnp.min(jnp.where(eqs[0], gidx[0], _IMAX), axis=1, keepdims=True)
            for l in range(1, nl):
                g = jnp.minimum(
                    g, jnp.min(jnp.where(eqs[l], gidx[l], _IMAX), axis=1,
                               keepdims=True))
            out_v.append(m)
            out_i.append(g)
            for l in range(nl):
                levels[l] = jnp.where(eqs[l] & (gidx[l] == g), _NEG, levels[l])
        outv_ref[:, :] = jnp.concatenate(out_v, axis=1)
        outi_ref[:, :] = jnp.concatenate(out_i, axis=1)


def kernel(query_encoding, entity_embedding):
    e_pad = jnp.pad(entity_embedding, ((0, _NB * _W - _NE), (0, 0)))
    vals, idx = pl.pallas_call(
        _topk_kernel,
        grid=(_B // _RB, _NB),
        in_specs=[
            pl.BlockSpec((_RB, _D), lambda i, j: (i, 0)),
            pl.BlockSpec((_W, _D), lambda i, j: (j, 0)),
        ],
        out_specs=(
            pl.BlockSpec((_RB, _K), lambda i, j: (i, 0)),
            pl.BlockSpec((_RB, _K), lambda i, j: (i, 0)),
        ),
        out_shape=(
            jax.ShapeDtypeStruct((_B, _K), jnp.float32),
            jax.ShapeDtypeStruct((_B, _K), jnp.int32),
        ),
        scratch_shapes=[
            pltpu.VMEM((_RB, _K * 128), jnp.float32),
            pltpu.VMEM((_RB, _K * 128), jnp.int32),
        ],
    )(query_encoding, e_pad)
    return (vals, idx)


# speedup vs baseline: 4.4763x; 1.0717x over previous
import jax
import jax.numpy as jnp
from jax.experimental import pallas as pl
from jax.experimental.pallas import tpu as pltpu

_B = 1024
_RB = 128
_SR = 8
_NS = _RB // _SR
_D = 128
_NE = 100000
_W = 2048
_NB = 49
_NC = _W // 128
_K = 10

_NEG = float("-inf")
_BIGF = float(2**25)


def _oddeven_sort_pairs(n):
    pairs = []

    def merge(lo, n_, r):
        step = r * 2
        if step < n_:
            merge(lo, n_, step)
            merge(lo + r, n_, step)
            for i in range(lo + r, lo + n_ - r, step):
                pairs.append((i, i + r))
        else:
            pairs.append((lo, lo + r))

    def sort(lo, n_):
        if n_ > 1:
            m = n_ // 2
            sort(lo, m)
            sort(lo + m, m)
            merge(lo, n_, 1)

    sort(0, n)
    return pairs


def _bitonic_merge16_pairs():
    pairs = []
    for d in (8, 4, 2, 1):
        for i in range(16):
            if (i // d) % 2 == 0 and i + d < 16:
                pairs.append((i, i + d))
    return pairs


def _prune(pairs, needed):
    need = set(needed)
    kept = []
    for (a, b) in reversed(pairs):
        if a in need or b in need:
            kept.append((a, b, b in need))
            need.add(a)
            need.add(b)
    kept.reverse()
    return kept


_SORT16 = _prune(_oddeven_sort_pairs(_NC), range(_K))
_MERGEK = _prune(_bitonic_merge16_pairs(), range(_K))


def _ce(v, ids, a, b, lo_needed):
    c = v[b] > v[a]
    hi = jnp.maximum(v[a], v[b])
    ihi = jnp.where(c, ids[b], ids[a])
    if lo_needed:
        lo = jnp.minimum(v[a], v[b])
        ilo = jnp.where(c, ids[a], ids[b])
        v[b], ids[b] = lo, ilo
    v[a], ids[a] = hi, ihi


def _topk_kernel(q_ref, e_ref, outv_ref, outi_ref, rv_ref, rn_ref, s_ref):
    j = pl.program_id(1)

    @pl.when(j == 0)
    def _init():
        rv_ref[:, :] = jnp.full((_RB, _K * 128), _NEG, jnp.float32)
        rn_ref[:, :] = jnp.zeros((_RB, _K * 128), jnp.int32)

    qb = q_ref[:, :].astype(jnp.bfloat16)
    eb = e_ref[:, :].astype(jnp.bfloat16)
    s_ref[:, :] = jax.lax.dot_general(qb, eb, (((1,), (1,)), ((), ())),
                                      preferred_element_type=jnp.float32)

    lane = jax.lax.broadcasted_iota(jnp.int32, (_SR, 128), 1)
    first_pad_chunk = (_NE - (_NB - 1) * _W) // 128
    ids0 = [jnp.full((_SR, 128), j * _NC + c, jnp.int32) for c in range(_NC)]
    padmask = {c: (j * _NC + c) * 128 + lane < _NE
               for c in range(first_pad_chunk, _NC)}

    for r in range(_NS):
        rs = slice(r * _SR, (r + 1) * _SR)
        v = []
        ids = list(ids0)
        for c in range(_NC):
            vc = s_ref[rs, c * 128:(c + 1) * 128]
            if c >= first_pad_chunk:
                vc = jnp.where(padmask[c], vc, _NEG)
            v.append(vc)

        for (a, b, ln) in _SORT16:
            _ce(v, ids, a, b, ln)

        w = []
        wi = []
        for i in range(_NC):
            bl = _NC - 1 - i
            if bl >= _K:
                w.append(v[i])
                wi.append(ids[i])
            else:
                bv = rv_ref[rs, bl * 128:(bl + 1) * 128]
                bn = rn_ref[rs, bl * 128:(bl + 1) * 128]
                c = bv > v[i]
                w.append(jnp.maximum(v[i], bv))
                wi.append(jnp.where(c, bn, ids[i]))
        for (a, b, ln) in _MERGEK:
            _ce(w, wi, a, b, ln)
        for i in range(_K):
            rv_ref[rs, i * 128:(i + 1) * 128] = w[i]
            rn_ref[rs, i * 128:(i + 1) * 128] = wi[i]

    @pl.when(j == _NB - 1)
    def _finalize():
        lanef = lane.astype(jnp.float32)
        for r in range(_NS):
            rs = slice(r * _SR, (r + 1) * _SR)
            levels = [rv_ref[rs, i * 128:(i + 1) * 128] for i in range(_K)]
            gidx = [rn_ref[rs, i * 128:(i + 1) * 128].astype(jnp.float32) * 128.0
                    + lanef for i in range(_K)]
            out_v = []
            out_i = []
            for k in range(_K):
                nl = k + 1
                m = levels[0].max(axis=1, keepdims=True)
                for l in range(1, nl):
                    m = jnp.maximum(m, levels[l].max(axis=1, keepdims=True))
                eqs = [levels[l] == m for l in range(nl)]
                g = jnp.min(jnp.where(eqs[0], gidx[0], _BIGF), axis=1,
                            keepdims=True)
                for l in range(1, nl):
                    g = jnp.minimum(
                        g, jnp.min(jnp.where(eqs[l], gidx[l], _BIGF), axis=1,
                                   keepdims=True))
                out_v.append(m)
                out_i.append(g)
                for l in range(nl):
                    levels[l] = jnp.where(eqs[l] & (gidx[l] == g), _NEG,
                                          levels[l])
            outv_ref[rs, :] = jnp.concatenate(out_v, axis=1)
            outi_ref[rs, :] = jnp.concatenate(out_i, axis=1).astype(jnp.int32)


def kernel(query_encoding, entity_embedding):
    e_pad = jnp.pad(entity_embedding, ((0, _NB * _W - _NE), (0, 0)))
    vals, idx = pl.pallas_call(
        _topk_kernel,
        grid=(_B // _RB, _NB),
        in_specs=[
            pl.BlockSpec((_RB, _D), lambda i, j: (i, 0)),
            pl.BlockSpec((_W, _D), lambda i, j: (j, 0)),
        ],
        out_specs=(
            pl.BlockSpec((_RB, _K), lambda i, j: (i, 0)),
            pl.BlockSpec((_RB, _K), lambda i, j: (i, 0)),
        ),
        out_shape=(
            jax.ShapeDtypeStruct((_B, _K), jnp.float32),
            jax.ShapeDtypeStruct((_B, _K), jnp.int32),
        ),
        scratch_shapes=[
            pltpu.VMEM((_RB, _K * 128), jnp.float32),
            pltpu.VMEM((_RB, _K * 128), jnp.int32),
            pltpu.VMEM((_RB, _W), jnp.float32),
        ],
    )(query_encoding, e_pad)
    return (vals, idx)
